# Initial kernel scaffold; baseline (speedup 1.0000x reference)
import jax
import jax.numpy as jnp
from jax.experimental import pallas as pl


def kernel(feature, flow):
    raise NotImplementedError("write your pallas kernel here")


# speedup vs baseline: 1.1173x; 1.1173x over previous
import functools

import jax
import jax.numpy as jnp
from jax import lax
from jax.experimental import pallas as pl
from jax.experimental.pallas import tpu as pltpu
from jax.experimental.pallas import tpu_sc as plsc


def _warp_sc(feat_rows, flow2, B, C, H, W):
    V = B * H * W
    info = plsc.get_sparse_core_info()
    NC, NS, L = info.num_cores, info.num_subcores, info.num_lanes
    NW = NC * NS
    T = 128
    assert C % L == 0 and V % NW == 0
    npix_w = V // NW
    assert npix_w % T == 0
    iters = npix_w // T
    per_batch = H * W
    assert per_batch % npix_w == 0
    assert W & (W - 1) == 0
    wshift = W.bit_length() - 1
    mesh = plsc.VectorSubcoreMesh(core_axis_name="c", subcore_axis_name="s")

    @functools.partial(
        pl.kernel,
        out_type=jax.ShapeDtypeStruct((V, C), jnp.float32),
        mesh=mesh,
        scratch_types=[
            pltpu.VMEM((T,), jnp.float32),
            pltpu.VMEM((T,), jnp.float32),
            pltpu.VMEM((4, T), jnp.int32),
            pltpu.VMEM((4, T), jnp.float32),
            pltpu.VMEM((4, T, C), jnp.float32),
            pltpu.VMEM((T, C), jnp.float32),
            pltpu.SemaphoreType.DMA,
        ],
        compiler_params=pltpu.CompilerParams(use_tc_tiling_on_sc=False),
    )
    def k(feat, flow, out, dx_v, dy_v, idx_v, w_v, rows_v, out_v, sem):
        cid = lax.axis_index("c")
        sid = lax.axis_index("s")
        wid = sid * NC + cid
        base = wid * npix_w
        b = base // per_batch
        q = base - b * per_batch
        bb = b * per_batch

        def chunk(i, carry):
            t0 = q + i * T
            pltpu.sync_copy(flow.at[2 * b, pl.ds(t0, T)], dx_v)
            pltpu.sync_copy(flow.at[2 * b + 1, pl.ds(t0, T)], dy_v)
            for g in range(T // L):
                sl = pl.ds(g * L, L)
                pp = t0 + g * L + lax.iota(jnp.int32, L)
                x = (pp & (W - 1)).astype(jnp.float32) + dx_v[sl]
                y = (pp >> wshift).astype(jnp.float32) + dy_v[sl]
                xt = x.astype(jnp.int32)
                xtf = xt.astype(jnp.float32)
                xneg = x < xtf
                x0i = jnp.where(xneg, xt - 1, xt)
                x0f = jnp.where(xneg, xtf - 1.0, xtf)
                yt = y.astype(jnp.int32)
                ytf = yt.astype(jnp.float32)
                yneg = y < ytf
                y0i = jnp.where(yneg, yt - 1, yt)
                y0f = jnp.where(yneg, ytf - 1.0, ytf)
                wx1 = x - x0f
                wx0 = 1.0 - wx1
                wy1 = y - y0f
                wy0 = 1.0 - wy1
                wx0 = jnp.where((x0f >= 0.0) & (x0f <= W - 1.0), wx0, 0.0)
                wx1 = jnp.where((x0f >= -1.0) & (x0f <= W - 2.0), wx1, 0.0)
                wy0 = jnp.where((y0f >= 0.0) & (y0f <= H - 1.0), wy0, 0.0)
                wy1 = jnp.where((y0f >= -1.0) & (y0f <= H - 2.0), wy1, 0.0)
                x0c = jnp.clip(x0i, 0, W - 1)
                x1c = jnp.clip(x0i + 1, 0, W - 1)
                r0 = bb + (jnp.clip(y0i, 0, H - 1) << wshift)
                r1 = bb + (jnp.clip(y0i + 1, 0, H - 1) << wshift)
                idx_v[0, sl] = r0 + x0c
                w_v[0, sl] = wy0 * wx0
                idx_v[1, sl] = r0 + x1c
                w_v[1, sl] = wy0 * wx1
                idx_v[2, sl] = r1 + x0c
                w_v[2, sl] = wy1 * wx0
                idx_v[3, sl] = r1 + x1c
                w_v[3, sl] = wy1 * wx1
            cps = [pltpu.async_copy(feat.at[idx_v.at[k_]], rows_v.at[k_], sem)
                   for k_ in range(4)]
            for cp in cps:
                cp.wait()

            def blend(g, c2):
                sl = pl.ds(g * L, L)
                w0v = w_v[0, sl]
                w1v = w_v[1, sl]
                w2v = w_v[2, sl]
                w3v = w_v[3, sl]
                for j in range(L):
                    w0 = w0v[j]
                    w1 = w1v[j]
                    w2 = w2v[j]
                    w3 = w3v[j]
                    t = g * L + j
                    for cg in range(C // L):
                        s = pl.ds(cg * L, L)
                        out_v[t, s] = (w0 * rows_v[0, t, s] + w1 * rows_v[1, t, s]
                                       + w2 * rows_v[2, t, s] + w3 * rows_v[3, t, s])
                return c2

            lax.fori_loop(0, T // L, blend, 0)
            pltpu.sync_copy(out_v, out.at[pl.ds(base + i * T, T)])
            return carry

        lax.fori_loop(0, iters, chunk, 0)

    return k(feat_rows, flow2)


def kernel(feature, flow):
    B, C, H, W = feature.shape
    feat_rows = jnp.transpose(feature, (0, 2, 3, 1)).reshape(B * H * W, C)
    flow2 = flow.reshape(B * 2, H * W)
    out_rows = _warp_sc(feat_rows, flow2, B, C, H, W)
    return out_rows.reshape(B, H, W, C).transpose(0, 3, 1, 2)


# speedup vs baseline: 1.2545x; 1.1228x over previous
import functools

import jax
import jax.numpy as jnp
from jax import lax
from jax.experimental import pallas as pl
from jax.experimental.pallas import tpu as pltpu
from jax.experimental.pallas import tpu_sc as plsc


def _warp_sc(feat_rows, flow2, B, C, H, W):
    V = B * H * W
    info = plsc.get_sparse_core_info()
    NC, NS, L = info.num_cores, info.num_subcores, info.num_lanes
    NW = NC * NS
    T = 64
    assert C % L == 0 and V % NW == 0
    npix_w = V // NW
    assert npix_w % T == 0
    iters = npix_w // T
    assert iters % 2 == 0 and iters >= 4
    per_batch = H * W
    assert per_batch % npix_w == 0
    assert W & (W - 1) == 0
    wshift = W.bit_length() - 1
    mesh = plsc.VectorSubcoreMesh(core_axis_name="c", subcore_axis_name="s")

    @functools.partial(
        pl.kernel,
        out_type=jax.ShapeDtypeStruct((V, C), jnp.float32),
        mesh=mesh,
        scratch_types=[
            pltpu.VMEM((2, T), jnp.float32),
            pltpu.VMEM((2, T), jnp.float32),
            pltpu.VMEM((2, 4, T), jnp.int32),
            pltpu.VMEM((2, 4, T), jnp.float32),
            pltpu.VMEM((2, 4, T, C), jnp.float32),
            pltpu.VMEM((2, T, C), jnp.float32),
            pltpu.SemaphoreType.DMA,
            pltpu.SemaphoreType.DMA,
            pltpu.SemaphoreType.DMA,
            pltpu.SemaphoreType.DMA,
        ],
        compiler_params=pltpu.CompilerParams(use_tc_tiling_on_sc=False),
    )
    def k(feat, flow, out, dx_v, dy_v, idx_v, w_v, rows_v, out_v,
          sem_g0, sem_g1, sem_o0, sem_o1):
        cid = lax.axis_index("c")
        sid = lax.axis_index("s")
        wid = sid * NC + cid
        base = wid * npix_w
        b = base // per_batch
        q = base - b * per_batch
        bb = b * per_batch
        sem_g = (sem_g0, sem_g1)
        sem_o = (sem_o0, sem_o1)

        def prep(j, nb):
            t0 = q + j * T
            pltpu.sync_copy(flow.at[2 * b, pl.ds(t0, T)], dx_v.at[nb])
            pltpu.sync_copy(flow.at[2 * b + 1, pl.ds(t0, T)], dy_v.at[nb])
            for g in range(T // L):
                sl = pl.ds(g * L, L)
                pp = t0 + g * L + lax.iota(jnp.int32, L)
                x = (pp & (W - 1)).astype(jnp.float32) + dx_v[nb, sl]
                y = (pp >> wshift).astype(jnp.float32) + dy_v[nb, sl]
                xt = x.astype(jnp.int32)
                xtf = xt.astype(jnp.float32)
                xneg = x < xtf
                x0i = jnp.where(xneg, xt - 1, xt)
                x0f = jnp.where(xneg, xtf - 1.0, xtf)
                yt = y.astype(jnp.int32)
                ytf = yt.astype(jnp.float32)
                yneg = y < ytf
                y0i = jnp.where(yneg, yt - 1, yt)
                y0f = jnp.where(yneg, ytf - 1.0, ytf)
                wx1 = x - x0f
                wx0 = 1.0 - wx1
                wy1 = y - y0f
                wy0 = 1.0 - wy1
                wx0 = jnp.where((x0f >= 0.0) & (x0f <= W - 1.0), wx0, 0.0)
                wx1 = jnp.where((x0f >= -1.0) & (x0f <= W - 2.0), wx1, 0.0)
                wy0 = jnp.where((y0f >= 0.0) & (y0f <= H - 1.0), wy0, 0.0)
                wy1 = jnp.where((y0f >= -1.0) & (y0f <= H - 2.0), wy1, 0.0)
                x0c = jnp.clip(x0i, 0, W - 1)
                x1c = jnp.clip(x0i + 1, 0, W - 1)
                r0 = bb + (jnp.clip(y0i, 0, H - 1) << wshift)
                r1 = bb + (jnp.clip(y0i + 1, 0, H - 1) << wshift)
                idx_v[nb, 0, sl] = r0 + x0c
                w_v[nb, 0, sl] = wy0 * wx0
                idx_v[nb, 1, sl] = r0 + x1c
                w_v[nb, 1, sl] = wy0 * wx1
                idx_v[nb, 2, sl] = r1 + x0c
                w_v[nb, 2, sl] = wy1 * wx0
                idx_v[nb, 3, sl] = r1 + x1c
                w_v[nb, 3, sl] = wy1 * wx1
            for k_ in range(4):
                pltpu.async_copy(feat.at[idx_v.at[nb, k_]], rows_v.at[nb, k_],
                                 sem_g[nb])

        def drain_gather(nb):
            for k_ in range(4):
                pltpu.make_async_copy(feat.at[pl.ds(0, T)], rows_v.at[nb, k_],
                                      sem_g[nb]).wait()

        def drain_store(nb):
            pltpu.make_async_copy(out_v.at[nb], out.at[pl.ds(0, T)],
                                  sem_o[nb]).wait()

        def blend_and_store(i, nb):
            def blend(g, c2):
                sl = pl.ds(g * L, L)
                w0v = w_v[nb, 0, sl]
                w1v = w_v[nb, 1, sl]
                w2v = w_v[nb, 2, sl]
                w3v = w_v[nb, 3, sl]
                for j in range(L):
                    w0 = w0v[j]
                    w1 = w1v[j]
                    w2 = w2v[j]
                    w3 = w3v[j]
                    t = g * L + j
                    for cg in range(C // L):
                        s = pl.ds(cg * L, L)
                        out_v[nb, t, s] = (
                            w0 * rows_v[nb, 0, t, s] + w1 * rows_v[nb, 1, t, s]
                            + w2 * rows_v[nb, 2, t, s] + w3 * rows_v[nb, 3, t, s])
                return c2

            lax.fori_loop(0, T // L, blend, 0)
            pltpu.async_copy(out_v.at[nb], out.at[pl.ds(base + i * T, T)],
                             sem_o[nb])

        prep(0, 0)

        def pair(s, carry):
            @pl.when(2 * s + 1 < iters)
            def _():
                prep(2 * s + 1, 1)
            drain_gather(0)
            @pl.when(s > 0)
            def _():
                drain_store(0)
            blend_and_store(2 * s, 0)
            @pl.when(2 * s + 2 < iters)
            def _():
                prep(2 * s + 2, 0)
            drain_gather(1)
            @pl.when(s > 0)
            def _():
                drain_store(1)
            blend_and_store(2 * s + 1, 1)
            return carry

        lax.fori_loop(0, iters // 2, pair, 0)
        drain_store(0)
        drain_store(1)

    return k(feat_rows, flow2)


def kernel(feature, flow):
    B, C, H, W = feature.shape
    feat_rows = jnp.transpose(feature, (0, 2, 3, 1)).reshape(B * H * W, C)
    flow2 = flow.reshape(B * 2, H * W)
    out_rows = _warp_sc(feat_rows, flow2, B, C, H, W)
    return out_rows.reshape(B, H, W, C).transpose(0, 3, 1, 2)
